# baseline (speedup 1.0000x reference)
import functools

import jax
import jax.numpy as jnp
from jax import lax
from jax.experimental import pallas as pl
from jax.experimental.pallas import tpu as pltpu
from jax.experimental.pallas import tpu_sc as plsc

_NC = 2
_NS = 16
_L = 16
_NW = _NC * _NS


def _body(users_h, items_h, uw_h, iw_h, ub_h, ib_h, out_h,
          uidx, iidx, urows, irows, ubv, ibv, outv,
          sem_uw, sem_iw, sem_ub, sem_ib, *, bpw, D):
    wid = lax.axis_index("s") * _NC + lax.axis_index("c")
    base = wid * bpw

    pltpu.sync_copy(users_h.at[pl.ds(base, bpw)], uidx)
    pltpu.sync_copy(items_h.at[pl.ds(base, bpw)], iidx)
    for c in range(bpw // _L):
        s = pl.ds(c * _L, _L)
        uidx[s] = uidx[s] - 1
        iidx[s] = iidx[s] - 1

    cp_uw = pltpu.async_copy(uw_h.at[uidx], urows, sem_uw)
    cp_iw = pltpu.async_copy(iw_h.at[iidx], irows, sem_iw)
    cp_ub = pltpu.async_copy(ub_h.at[uidx], ubv, sem_ub)
    cp_ib = pltpu.async_copy(ib_h.at[iidx], ibv, sem_ib)
    cp_uw.wait()
    cp_iw.wait()
    cp_ub.wait()
    cp_ib.wait()

    col0 = lax.iota(jnp.int32, _L)

    def group(g, carry):
        b0 = g * _L
        row16 = col0 + b0
        acc = ubv[pl.ds(b0, _L)] + ibv[pl.ds(b0, _L)]
        for d in range(D):
            colv = lax.bitwise_and(col0 + d, D - 1)
            uv = plsc.load_gather(urows, [row16, colv])
            iv = plsc.load_gather(irows, [row16, colv])
            acc = acc + uv * iv
        outv[pl.ds(b0, _L)] = 5.0 / (1.0 + jnp.exp(-acc))
        return carry

    lax.fori_loop(0, bpw // _L, group, 0)

    pltpu.sync_copy(outv, out_h.at[pl.ds(base, bpw)])


@jax.jit
def kernel(users, items, u_weight, i_weight, u_bias, i_bias):
    B = users.shape[0]
    D = u_weight.shape[1]
    bpw = B // _NW
    mesh = plsc.VectorSubcoreMesh(core_axis_name="c", subcore_axis_name="s")
    f = pl.kernel(
        functools.partial(_body, bpw=bpw, D=D),
        out_type=jax.ShapeDtypeStruct((B,), jnp.float32),
        mesh=mesh,
        compiler_params=pltpu.CompilerParams(
            needs_layout_passes=False, use_tc_tiling_on_sc=False),
        scratch_types=[
            pltpu.VMEM((bpw,), jnp.int32),
            pltpu.VMEM((bpw,), jnp.int32),
            pltpu.VMEM((bpw, D), jnp.float32),
            pltpu.VMEM((bpw, D), jnp.float32),
            pltpu.VMEM((bpw,), jnp.float32),
            pltpu.VMEM((bpw,), jnp.float32),
            pltpu.VMEM((bpw,), jnp.float32),
            pltpu.SemaphoreType.DMA,
            pltpu.SemaphoreType.DMA,
            pltpu.SemaphoreType.DMA,
            pltpu.SemaphoreType.DMA,
        ],
    )
    return f(users, items, u_weight, i_weight,
             u_bias.reshape(-1), i_bias.reshape(-1))
